# baseline (speedup 1.0000x reference)
import functools

import jax
import jax.numpy as jnp
from jax.experimental import pallas as pl


def _rvq_block(z_ref, cbbf_ref, hcb2_ref, hi_ref, mid_ref, lo_ref,
               zq_ref, idx_ref, *, n_cb, K, cd):
    R = z_ref.shape[0]
    iota0 = jax.lax.broadcasted_iota(jnp.int32, (K, R), 0)
    carry = jnp.zeros((R, cd), dtype=jnp.float32)
    for i in range(n_cb):
        res = z_ref[:, i * cd:(i + 1) * cd] + carry
        mm = jax.lax.dot_general(
            cbbf_ref[i], res.astype(jnp.bfloat16),
            (((1,), (1,)), ((), ())),
            preferred_element_type=jnp.float32)
        scores = hcb2_ref[i] - mm
        m = jnp.min(scores, axis=0, keepdims=True)
        idx = jnp.min(jnp.where(scores == m, iota0, K), axis=0, keepdims=True)
        onehot = (iota0 == idx).astype(jnp.bfloat16)
        qt = (jax.lax.dot_general(hi_ref[i], onehot, (((1,), (0,)), ((), ())),
                                  preferred_element_type=jnp.float32)
              + jax.lax.dot_general(mid_ref[i], onehot, (((1,), (0,)), ((), ())),
                                    preferred_element_type=jnp.float32)) \
            + jax.lax.dot_general(lo_ref[i], onehot, (((1,), (0,)), ((), ())),
                                  preferred_element_type=jnp.float32)
        q = qt.T
        zq_ref[:, i * cd:(i + 1) * cd] = q
        idx_ref[i:i + 1, :] = idx
        if i < n_cb - 1:
            carry = res - q


@functools.partial(jax.jit, static_argnames=())
def kernel(z, codebooks):
    B, T, D = z.shape
    n_cb, K, cd = codebooks.shape
    rows = B * T
    R = 512
    zf = z.reshape(rows, D)
    cb_bf = codebooks.astype(jnp.bfloat16)
    hcb2 = 0.5 * jnp.sum(codebooks * codebooks, axis=-1)[..., None]
    hi = jax.lax.optimization_barrier(codebooks.astype(jnp.bfloat16))
    r1 = codebooks - hi.astype(jnp.float32)
    mid = jax.lax.optimization_barrier(r1.astype(jnp.bfloat16))
    lo = (r1 - mid.astype(jnp.float32)).astype(jnp.bfloat16)
    hiT = jnp.swapaxes(hi, 1, 2)
    midT = jnp.swapaxes(mid, 1, 2)
    loT = jnp.swapaxes(lo, 1, 2)

    zq_flat, idx8 = pl.pallas_call(
        functools.partial(_rvq_block, n_cb=n_cb, K=K, cd=cd),
        grid=(rows // R,),
        in_specs=[
            pl.BlockSpec((R, D), lambda b: (b, 0)),
            pl.BlockSpec((n_cb, K, cd), lambda b: (0, 0, 0)),
            pl.BlockSpec((n_cb, K, 1), lambda b: (0, 0, 0)),
            pl.BlockSpec((n_cb, cd, K), lambda b: (0, 0, 0)),
            pl.BlockSpec((n_cb, cd, K), lambda b: (0, 0, 0)),
            pl.BlockSpec((n_cb, cd, K), lambda b: (0, 0, 0)),
        ],
        out_specs=[
            pl.BlockSpec((R, D), lambda b: (b, 0)),
            pl.BlockSpec((n_cb, R), lambda b: (0, b)),
        ],
        out_shape=[
            jax.ShapeDtypeStruct((rows, D), jnp.float32),
            jax.ShapeDtypeStruct((n_cb, rows), jnp.int32),
        ],
    )(zf, cb_bf, hcb2, hiT, midT, loT)

    z_q = zq_flat.reshape(B, T, D)
    indices = idx8.reshape(n_cb, B, T).transpose(1, 0, 2)
    return (z_q, indices)


# speedup vs baseline: 1.2859x; 1.2859x over previous
import functools

import jax
import jax.numpy as jnp
from jax.experimental import pallas as pl


def _rvq_block(z_ref, cbbf_ref, hcb2_ref, hi_ref, mid_ref, lo_ref,
               zq_ref, idx_ref, *, n_cb, K, cd, n_half):
    R = z_ref.shape[0]
    H = R // n_half
    iota0 = jax.lax.broadcasted_iota(jnp.int32, (K, H), 0)
    carries = [jnp.zeros((H, cd), dtype=jnp.float32) for _ in range(n_half)]
    for i in range(n_cb):
        for h in range(n_half):
            r0 = h * H
            res = z_ref[r0:r0 + H, i * cd:(i + 1) * cd] + carries[h]
            mm = jax.lax.dot_general(
                cbbf_ref[i], res.astype(jnp.bfloat16),
                (((1,), (1,)), ((), ())),
                preferred_element_type=jnp.float32)
            scores = hcb2_ref[i] - mm
            m = jnp.min(scores, axis=0, keepdims=True)
            idx = jnp.min(jnp.where(scores == m, iota0, K),
                          axis=0, keepdims=True)
            onehot = (iota0 == idx).astype(jnp.bfloat16)
            qt = (jax.lax.dot_general(hi_ref[i], onehot,
                                      (((1,), (0,)), ((), ())),
                                      preferred_element_type=jnp.float32)
                  + jax.lax.dot_general(mid_ref[i], onehot,
                                        (((1,), (0,)), ((), ())),
                                        preferred_element_type=jnp.float32)) \
                + jax.lax.dot_general(lo_ref[i], onehot,
                                      (((1,), (0,)), ((), ())),
                                      preferred_element_type=jnp.float32)
            q = qt.T
            zq_ref[r0:r0 + H, i * cd:(i + 1) * cd] = q
            idx_ref[i:i + 1, r0:r0 + H] = idx
            if i < n_cb - 1:
                carries[h] = res - q


@functools.partial(jax.jit, static_argnames=())
def kernel(z, codebooks):
    B, T, D = z.shape
    n_cb, K, cd = codebooks.shape
    rows = B * T
    R = 1024
    zf = z.reshape(rows, D)
    cb_bf = codebooks.astype(jnp.bfloat16)
    hcb2 = 0.5 * jnp.sum(codebooks * codebooks, axis=-1)[..., None]
    hi = jax.lax.optimization_barrier(codebooks.astype(jnp.bfloat16))
    r1 = codebooks - hi.astype(jnp.float32)
    mid = jax.lax.optimization_barrier(r1.astype(jnp.bfloat16))
    lo = (r1 - mid.astype(jnp.float32)).astype(jnp.bfloat16)
    hiT = jnp.swapaxes(hi, 1, 2)
    midT = jnp.swapaxes(mid, 1, 2)
    loT = jnp.swapaxes(lo, 1, 2)

    zq_flat, idx8 = pl.pallas_call(
        functools.partial(_rvq_block, n_cb=n_cb, K=K, cd=cd, n_half=2),
        grid=(rows // R,),
        in_specs=[
            pl.BlockSpec((R, D), lambda b: (b, 0)),
            pl.BlockSpec((n_cb, K, cd), lambda b: (0, 0, 0)),
            pl.BlockSpec((n_cb, K, 1), lambda b: (0, 0, 0)),
            pl.BlockSpec((n_cb, cd, K), lambda b: (0, 0, 0)),
            pl.BlockSpec((n_cb, cd, K), lambda b: (0, 0, 0)),
            pl.BlockSpec((n_cb, cd, K), lambda b: (0, 0, 0)),
        ],
        out_specs=[
            pl.BlockSpec((R, D), lambda b: (b, 0)),
            pl.BlockSpec((n_cb, R), lambda b: (0, b)),
        ],
        out_shape=[
            jax.ShapeDtypeStruct((rows, D), jnp.float32),
            jax.ShapeDtypeStruct((n_cb, rows), jnp.int32),
        ],
    )(zf, cb_bf, hcb2, hiT, midT, loT)

    z_q = zq_flat.reshape(B, T, D)
    indices = idx8.reshape(n_cb, B, T).transpose(1, 0, 2)
    return (z_q, indices)


# speedup vs baseline: 1.4930x; 1.1610x over previous
import functools

import jax
import jax.numpy as jnp
from jax.experimental import pallas as pl


def _rvq_block(z_ref, cbbf_ref, hcb2_ref, hi_ref, mid_ref, lo_ref,
               zq_ref, idx_ref, *, n_cb, K, cd, n_half):
    R = z_ref.shape[0]
    H = R // n_half
    iota0 = jax.lax.broadcasted_iota(jnp.int32, (K, H), 0)
    carries = [jnp.zeros((H, cd), dtype=jnp.float32) for _ in range(n_half)]
    for i in range(n_cb):
        for h in range(n_half):
            r0 = h * H
            res = z_ref[r0:r0 + H, i * cd:(i + 1) * cd] + carries[h]
            mm = jax.lax.dot_general(
                cbbf_ref[i], res.astype(jnp.bfloat16),
                (((1,), (1,)), ((), ())),
                preferred_element_type=jnp.float32)
            scores = hcb2_ref[i] - mm
            idx = jnp.argmin(scores, axis=0)[None, :].astype(jnp.int32)
            onehot = (iota0 == idx).astype(jnp.bfloat16)
            qt = (jax.lax.dot_general(hi_ref[i], onehot,
                                      (((1,), (0,)), ((), ())),
                                      preferred_element_type=jnp.float32)
                  + jax.lax.dot_general(mid_ref[i], onehot,
                                        (((1,), (0,)), ((), ())),
                                        preferred_element_type=jnp.float32)) \
                + jax.lax.dot_general(lo_ref[i], onehot,
                                      (((1,), (0,)), ((), ())),
                                      preferred_element_type=jnp.float32)
            q = qt.T
            zq_ref[r0:r0 + H, i * cd:(i + 1) * cd] = q
            idx_ref[i:i + 1, r0:r0 + H] = idx
            if i < n_cb - 1:
                carries[h] = res - q


@functools.partial(jax.jit, static_argnames=())
def kernel(z, codebooks):
    B, T, D = z.shape
    n_cb, K, cd = codebooks.shape
    rows = B * T
    R = 1024
    zf = z.reshape(rows, D)
    cb_bf = codebooks.astype(jnp.bfloat16)
    hcb2 = 0.5 * jnp.sum(codebooks * codebooks, axis=-1)[..., None]
    hi = jax.lax.optimization_barrier(codebooks.astype(jnp.bfloat16))
    r1 = codebooks - hi.astype(jnp.float32)
    mid = jax.lax.optimization_barrier(r1.astype(jnp.bfloat16))
    lo = (r1 - mid.astype(jnp.float32)).astype(jnp.bfloat16)
    hiT = jnp.swapaxes(hi, 1, 2)
    midT = jnp.swapaxes(mid, 1, 2)
    loT = jnp.swapaxes(lo, 1, 2)

    zq_flat, idx8 = pl.pallas_call(
        functools.partial(_rvq_block, n_cb=n_cb, K=K, cd=cd, n_half=2),
        grid=(rows // R,),
        in_specs=[
            pl.BlockSpec((R, D), lambda b: (b, 0)),
            pl.BlockSpec((n_cb, K, cd), lambda b: (0, 0, 0)),
            pl.BlockSpec((n_cb, K, 1), lambda b: (0, 0, 0)),
            pl.BlockSpec((n_cb, cd, K), lambda b: (0, 0, 0)),
            pl.BlockSpec((n_cb, cd, K), lambda b: (0, 0, 0)),
            pl.BlockSpec((n_cb, cd, K), lambda b: (0, 0, 0)),
        ],
        out_specs=[
            pl.BlockSpec((R, D), lambda b: (b, 0)),
            pl.BlockSpec((n_cb, R), lambda b: (0, b)),
        ],
        out_shape=[
            jax.ShapeDtypeStruct((rows, D), jnp.float32),
            jax.ShapeDtypeStruct((n_cb, rows), jnp.int32),
        ],
    )(zf, cb_bf, hcb2, hiT, midT, loT)

    z_q = zq_flat.reshape(B, T, D)
    indices = idx8.reshape(n_cb, B, T).transpose(1, 0, 2)
    return (z_q, indices)


# speedup vs baseline: 2.1639x; 1.4494x over previous
import functools

import jax
import jax.numpy as jnp
from jax.experimental import pallas as pl


def _rvq_block(z_ref, cbbf_ref, hcb2_ref, hml_ref,
               zq_ref, idx_ref, *, n_cb, K, cd, n_half):
    R = z_ref.shape[0]
    H = R // n_half
    iota0 = jax.lax.broadcasted_iota(jnp.int32, (K, H), 0)
    carries = [jnp.zeros((H, cd), dtype=jnp.float32) for _ in range(n_half)]
    for i in range(n_cb):
        for h in range(n_half):
            r0 = h * H
            res = z_ref[r0:r0 + H, i * cd:(i + 1) * cd] + carries[h]
            mm = jax.lax.dot_general(
                cbbf_ref[i], res.astype(jnp.bfloat16),
                (((1,), (1,)), ((), ())),
                preferred_element_type=jnp.float32)
            scores = hcb2_ref[i] - mm
            idx = jnp.argmin(scores, axis=0)[None, :].astype(jnp.int32)
            onehot = (iota0 == idx).astype(jnp.bfloat16)
            qcat = jax.lax.dot_general(hml_ref[i], onehot,
                                       (((1,), (0,)), ((), ())),
                                       preferred_element_type=jnp.float32)
            qt = (qcat[0:cd] + qcat[cd:2 * cd]) + qcat[2 * cd:3 * cd]
            q = qt.T
            zq_ref[r0:r0 + H, i * cd:(i + 1) * cd] = q
            idx_ref[i:i + 1, r0:r0 + H] = idx
            if i < n_cb - 1:
                carries[h] = res - q


@functools.partial(jax.jit, static_argnames=())
def kernel(z, codebooks):
    B, T, D = z.shape
    n_cb, K, cd = codebooks.shape
    rows = B * T
    R = 1024
    zf = z.reshape(rows, D)
    cb_bf = codebooks.astype(jnp.bfloat16)
    hcb2 = 0.5 * jnp.sum(codebooks * codebooks, axis=-1)[..., None]
    hi = jax.lax.optimization_barrier(codebooks.astype(jnp.bfloat16))
    r1 = codebooks - hi.astype(jnp.float32)
    mid = jax.lax.optimization_barrier(r1.astype(jnp.bfloat16))
    lo = (r1 - mid.astype(jnp.float32)).astype(jnp.bfloat16)
    hmlT = jnp.concatenate([jnp.swapaxes(hi, 1, 2),
                            jnp.swapaxes(mid, 1, 2),
                            jnp.swapaxes(lo, 1, 2)], axis=1)

    zq_flat, idx8 = pl.pallas_call(
        functools.partial(_rvq_block, n_cb=n_cb, K=K, cd=cd, n_half=2),
        grid=(rows // R,),
        in_specs=[
            pl.BlockSpec((R, D), lambda b: (b, 0)),
            pl.BlockSpec((n_cb, K, cd), lambda b: (0, 0, 0)),
            pl.BlockSpec((n_cb, K, 1), lambda b: (0, 0, 0)),
            pl.BlockSpec((n_cb, 3 * cd, K), lambda b: (0, 0, 0)),
        ],
        out_specs=[
            pl.BlockSpec((R, D), lambda b: (b, 0)),
            pl.BlockSpec((n_cb, R), lambda b: (0, b)),
        ],
        out_shape=[
            jax.ShapeDtypeStruct((rows, D), jnp.float32),
            jax.ShapeDtypeStruct((n_cb, rows), jnp.int32),
        ],
    )(zf, cb_bf, hcb2, hmlT)

    z_q = zq_flat.reshape(B, T, D)
    indices = idx8.reshape(n_cb, B, T).transpose(1, 0, 2)
    return (z_q, indices)
